# Initial kernel scaffold; baseline (speedup 1.0000x reference)
import jax
import jax.numpy as jnp
from jax.experimental import pallas as pl


def kernel(gt_depths, ModFs, DemodFs):
    raise NotImplementedError("write your pallas kernel here")


# speedup vs baseline: 183.6687x; 183.6687x over previous
import functools
import math

import numpy as np
import jax
import jax.numpy as jnp
from jax import lax
from jax.experimental import pallas as pl
from jax.experimental.pallas import tpu as pltpu
from jax.experimental.pallas import tpu_sc as plsc

N_CODE = 10000
FCT = 100
K = 3
SPEED_OF_LIGHT = 299792458.0 * 1000.0
D_MAX = 10000.0
F_MAX = SPEED_OF_LIGHT / (2.0 * D_MAX)
TAU_MIN = 1.0 / F_MAX
DT = TAU_MIN / float(N_CODE)
P_AMBIENT = float(10 ** 6)

_ar = np.arange(FCT)
_DC = np.cos(2.0 * np.pi * np.outer(_ar, _ar) / FCT).astype(np.float32)
_DS = np.sin(2.0 * np.pi * np.outer(_ar, _ar) / FCT).astype(np.float32)
_WC = np.cos(2.0 * np.pi * np.outer(_ar, _ar) / N_CODE).astype(np.float32)
_WS = np.sin(2.0 * np.pi * np.outer(_ar, _ar) / N_CODE).astype(np.float32)

P_BLK = 256
N_PIX = 4096
N_CHUNK = 2000


def _dot(a, b):
    return jnp.dot(a, b, preferred_element_type=jnp.float32)


def _prep_body(m_ref, d_ref, dc_ref, ds_ref, wc_ref, ws_ref, ncf_ref, tab_ref):
    Dc = dc_ref[...]
    Ds = ds_ref[...]
    Wc = wc_ref[...]
    Ws = ws_ref[...]

    def fwd(Xt):
        Gr = _dot(Xt, Dc)
        Gi = -_dot(Xt, Ds)
        Hr = Gr * Wc + Gi * Ws
        Hi = Gi * Wc - Gr * Ws
        Fr = _dot(Dc, Hr) + _dot(Ds, Hi)
        Fi = _dot(Dc, Hi) - _dot(Ds, Hr)
        return Fr, Fi

    for j in range(K):
        Xt = m_ref[j]
        Yt = d_ref[j]
        FMr, FMi = fwd(Xt)
        FDr, FDi = fwd(Yt)
        Pr = FMr * FDr + FMi * FDi
        Pi = FMr * FDi - FMi * FDr
        Ar = _dot(Dc, Pr) - _dot(Ds, Pi)
        Ai = _dot(Dc, Pi) + _dot(Ds, Pr)
        Br = Ar * Wc - Ai * Ws
        Bi = Ar * Ws + Ai * Wc
        RmT = (_dot(Br, Dc) - _dot(Bi, Ds)) * (1.0 / N_CODE)
        mu = jnp.mean(RmT)
        dev = RmT - mu
        sd = jnp.sqrt(jnp.sum(dev * dev) * (1.0 / (N_CODE - 1)))
        ncf_ref[j] = dev / sd
        kappa = jnp.sum(Yt) * DT
        tab_ref[j] = RmT * DT + P_AMBIENT * kappa


def _sc_gather_make(nw, ppw, lanes):
    mesh = plsc.VectorSubcoreMesh(core_axis_name="c", subcore_axis_name="s")
    nc = plsc.get_sparse_core_info().num_cores

    @functools.partial(
        pl.kernel,
        mesh=mesh,
        out_type=jax.ShapeDtypeStruct((K, N_PIX), jnp.float32),
        scratch_types=[
            pltpu.VMEM((ppw,), jnp.float32),
            pltpu.VMEM((K, ppw), jnp.int32),
            pltpu.VMEM((K, ppw), jnp.float32),
            pltpu.SemaphoreType.DMA,
        ],
    )
    def sc_gather(gt_hbm, tab_hbm, out_hbm, gt_v, idx_v, val_v, sem):
        wid = lax.axis_index("s") * nc + lax.axis_index("c")
        base = wid * ppw
        pltpu.sync_copy(gt_hbm.at[pl.ds(base, ppw)], gt_v)
        for t in range(ppw // lanes):
            g = gt_v[pl.ds(t * lanes, lanes)]
            r = g + 0.5
            ri = r.astype(jnp.int32)
            rf = ri.astype(jnp.float32)
            fix = (rf == r) & ((ri & 1) == 1)
            ri = jnp.where(fix, ri - 1, ri)
            ri = jnp.clip(ri, 0, N_CODE - 1)
            fl = ri * K
            for col in range(K):
                idx_v[col, pl.ds(t * lanes, lanes)] = fl + col
        for col in range(K):
            pltpu.async_copy(tab_hbm.at[idx_v.at[col]], val_v.at[col], sem).wait()
        pltpu.sync_copy(val_v, out_hbm.at[:, pl.ds(base, ppw)])

    return sc_gather


def _decode_body(bv_ref, ncf_ref, out_ref):
    v = bv_ref[...]
    mu = jnp.mean(v, axis=1, keepdims=True)
    dev = v - mu
    sd = jnp.sqrt(jnp.sum(dev * dev, axis=1, keepdims=True) * (1.0 / (K - 1)))
    nb = dev / sd
    nb0 = nb[:, 0:1]
    nb1 = nb[:, 1:2]
    nb2 = nb[:, 2:3]
    s0 = jnp.zeros((P_BLK, 1), jnp.float32)
    s1 = jnp.zeros((P_BLK, 1), jnp.float32)
    for k in range(N_CODE // N_CHUNK):
        r0 = ncf_ref[0:1, pl.ds(k * N_CHUNK, N_CHUNK)]
        r1 = ncf_ref[1:2, pl.ds(k * N_CHUNK, N_CHUNK)]
        r2 = ncf_ref[2:3, pl.ds(k * N_CHUNK, N_CHUNK)]
        C = nb0 * r0 + nb1 * r1 + nb2 * r2
        E = jnp.exp(C)
        w = lax.broadcasted_iota(jnp.int32, (1, N_CHUNK), 1).astype(jnp.float32) + float(k * N_CHUNK)
        s0 = s0 + jnp.sum(E, axis=1, keepdims=True)
        s1 = s1 + jnp.sum(E * w, axis=1, keepdims=True)
    out_ref[...] = s1 / s0


def kernel(gt_depths, ModFs, DemodFs):
    f32 = jnp.float32
    Mm = jnp.transpose(ModFs.astype(f32)).reshape(K, FCT, FCT).transpose(0, 2, 1)
    Dm = jnp.transpose(DemodFs.astype(f32)).reshape(K, FCT, FCT).transpose(0, 2, 1)

    ncf3, tab3 = pl.pallas_call(
        _prep_body,
        out_shape=(
            jax.ShapeDtypeStruct((K, FCT, FCT), f32),
            jax.ShapeDtypeStruct((K, FCT, FCT), f32),
        ),
    )(Mm, Dm, jnp.asarray(_DC), jnp.asarray(_DS), jnp.asarray(_WC), jnp.asarray(_WS))

    ncf = ncf3.transpose(0, 2, 1).reshape(K, N_CODE)
    tab_flat = tab3.transpose(2, 1, 0).reshape(N_CODE * K)

    info = plsc.get_sparse_core_info()
    nw = info.num_cores * info.num_subcores
    ppw = N_PIX // nw
    gt_flat = gt_depths.astype(f32).reshape(N_PIX)
    bv_cols = _sc_gather_make(nw, ppw, info.num_lanes)(gt_flat, tab_flat)
    bv = jnp.transpose(bv_cols)

    dec = pl.pallas_call(
        _decode_body,
        grid=(N_PIX // P_BLK,),
        in_specs=[
            pl.BlockSpec((P_BLK, K), lambda i: (i, 0)),
            pl.BlockSpec((K, N_CODE), lambda i: (0, 0)),
        ],
        out_specs=pl.BlockSpec((P_BLK, 1), lambda i: (i, 0)),
        out_shape=jax.ShapeDtypeStruct((N_PIX, 1), f32),
    )(bv, ncf)

    return dec.reshape(gt_depths.shape)


# speedup vs baseline: 261.6819x; 1.4247x over previous
import functools
import math

import numpy as np
import jax
import jax.numpy as jnp
from jax import lax
from jax.experimental import pallas as pl
from jax.experimental.pallas import tpu as pltpu
from jax.experimental.pallas import tpu_sc as plsc

N_CODE = 10000
FCT = 100
K = 3
SPEED_OF_LIGHT = 299792458.0 * 1000.0
D_MAX = 10000.0
F_MAX = SPEED_OF_LIGHT / (2.0 * D_MAX)
TAU_MIN = 1.0 / F_MAX
DT = TAU_MIN / float(N_CODE)
P_AMBIENT = float(10 ** 6)

_ar = np.arange(FCT)
_DC = np.cos(2.0 * np.pi * np.outer(_ar, _ar) / FCT).astype(np.float32)
_DS = np.sin(2.0 * np.pi * np.outer(_ar, _ar) / FCT).astype(np.float32)
_WC = np.cos(2.0 * np.pi * np.outer(_ar, _ar) / N_CODE).astype(np.float32)
_WS = np.sin(2.0 * np.pi * np.outer(_ar, _ar) / N_CODE).astype(np.float32)

P_BLK = 256
N_PIX = 4096
N_CHUNK = 2000

_W2T = np.stack(
    [np.ones(N_CODE, np.float32), np.arange(N_CODE, dtype=np.float32)], axis=0
)


def _dot(a, b):
    return jnp.dot(a, b, preferred_element_type=jnp.float32)


def _prep_body(m_ref, d_ref, dc_ref, ds_ref, wc_ref, ws_ref, ncf_ref, tab_ref):
    Dc = dc_ref[...]
    Ds = ds_ref[...]
    Wc = wc_ref[...]
    Ws = ws_ref[...]

    def fwd(Xt):
        Gr = _dot(Xt, Dc)
        Gi = -_dot(Xt, Ds)
        Hr = Gr * Wc + Gi * Ws
        Hi = Gi * Wc - Gr * Ws
        Fr = _dot(Dc, Hr) + _dot(Ds, Hi)
        Fi = _dot(Dc, Hi) - _dot(Ds, Hr)
        return Fr, Fi

    for j in range(K):
        Xt = m_ref[j]
        Yt = d_ref[j]
        FMr, FMi = fwd(Xt)
        FDr, FDi = fwd(Yt)
        Pr = FMr * FDr + FMi * FDi
        Pi = FMr * FDi - FMi * FDr
        Ar = _dot(Dc, Pr) - _dot(Ds, Pi)
        Ai = _dot(Dc, Pi) + _dot(Ds, Pr)
        Br = Ar * Wc - Ai * Ws
        Bi = Ar * Ws + Ai * Wc
        RmT = (_dot(Br, Dc) - _dot(Bi, Ds)) * (1.0 / N_CODE)
        mu = jnp.mean(RmT)
        dev = RmT - mu
        sd = jnp.sqrt(jnp.sum(dev * dev) * (1.0 / (N_CODE - 1)))
        ncf_ref[j] = dev / sd
        kappa = jnp.sum(Yt) * DT
        tab_ref[j] = RmT * DT + P_AMBIENT * kappa


def _sc_gather_make(nw, ppw, lanes):
    mesh = plsc.VectorSubcoreMesh(core_axis_name="c", subcore_axis_name="s")
    nc = plsc.get_sparse_core_info().num_cores

    @functools.partial(
        pl.kernel,
        mesh=mesh,
        out_type=jax.ShapeDtypeStruct((K, N_PIX), jnp.float32),
        scratch_types=[
            pltpu.VMEM((ppw,), jnp.float32),
            pltpu.VMEM((K, ppw), jnp.int32),
            pltpu.VMEM((K, ppw), jnp.float32),
            pltpu.SemaphoreType.DMA,
        ],
    )
    def sc_gather(gt_hbm, tab_hbm, out_hbm, gt_v, idx_v, val_v, sem):
        wid = lax.axis_index("s") * nc + lax.axis_index("c")
        base = wid * ppw
        pltpu.sync_copy(gt_hbm.at[pl.ds(base, ppw)], gt_v)
        for t in range(ppw // lanes):
            g = gt_v[pl.ds(t * lanes, lanes)]
            r = g + 0.5
            ri = r.astype(jnp.int32)
            rf = ri.astype(jnp.float32)
            fix = (rf == r) & ((ri & 1) == 1)
            ri = jnp.where(fix, ri - 1, ri)
            ri = jnp.clip(ri, 0, N_CODE - 1)
            fl = ri * K
            for col in range(K):
                idx_v[col, pl.ds(t * lanes, lanes)] = fl + col
        for col in range(K):
            pltpu.async_copy(tab_hbm.at[idx_v.at[col]], val_v.at[col], sem).wait()
        pltpu.sync_copy(val_v, out_hbm.at[:, pl.ds(base, ppw)])

    return sc_gather


def _decode_body(bv_ref, ncf_ref, w2_ref, out_ref):
    v = bv_ref[...]
    mu = jnp.mean(v, axis=1, keepdims=True)
    dev = v - mu
    sd = jnp.sqrt(jnp.sum(dev * dev, axis=1, keepdims=True) * (1.0 / (K - 1)))
    nb = dev / sd
    S = jnp.zeros((P_BLK, 2), jnp.float32)
    for k in range(N_CODE // N_CHUNK):
        ncf_c = ncf_ref[:, pl.ds(k * N_CHUNK, N_CHUNK)]
        C = _dot(nb, ncf_c)
        E = jnp.exp(C)
        w2_c = w2_ref[:, pl.ds(k * N_CHUNK, N_CHUNK)]
        S = S + lax.dot_general(
            E, w2_c, (((1,), (1,)), ((), ())), preferred_element_type=jnp.float32
        )
    out_ref[...] = S[:, 1:2] / S[:, 0:1]


def kernel(gt_depths, ModFs, DemodFs):
    f32 = jnp.float32
    Mm = jnp.transpose(ModFs.astype(f32)).reshape(K, FCT, FCT).transpose(0, 2, 1)
    Dm = jnp.transpose(DemodFs.astype(f32)).reshape(K, FCT, FCT).transpose(0, 2, 1)

    ncf3, tab3 = pl.pallas_call(
        _prep_body,
        out_shape=(
            jax.ShapeDtypeStruct((K, FCT, FCT), f32),
            jax.ShapeDtypeStruct((K, FCT, FCT), f32),
        ),
    )(Mm, Dm, jnp.asarray(_DC), jnp.asarray(_DS), jnp.asarray(_WC), jnp.asarray(_WS))

    ncf = ncf3.transpose(0, 2, 1).reshape(K, N_CODE)
    tab_flat = tab3.transpose(2, 1, 0).reshape(N_CODE * K)

    info = plsc.get_sparse_core_info()
    nw = info.num_cores * info.num_subcores
    ppw = N_PIX // nw
    gt_flat = gt_depths.astype(f32).reshape(N_PIX)
    bv_cols = _sc_gather_make(nw, ppw, info.num_lanes)(gt_flat, tab_flat)
    bv = jnp.transpose(bv_cols)

    dec = pl.pallas_call(
        _decode_body,
        grid=(N_PIX // P_BLK,),
        in_specs=[
            pl.BlockSpec((P_BLK, K), lambda i: (i, 0)),
            pl.BlockSpec((K, N_CODE), lambda i: (0, 0)),
            pl.BlockSpec((2, N_CODE), lambda i: (0, 0)),
        ],
        out_specs=pl.BlockSpec((P_BLK, 1), lambda i: (i, 0)),
        out_shape=jax.ShapeDtypeStruct((N_PIX, 1), f32),
    )(bv, ncf, jnp.asarray(_W2T))

    return dec.reshape(gt_depths.shape)
